# baseline (speedup 1.0000x reference)
import functools

import jax
import jax.numpy as jnp
from jax import lax
from jax.experimental import pallas as pl
from jax.experimental.pallas import tpu as pltpu
from jax.experimental.pallas import tpu_sc as plsc

_NUM_CLASSES = 100000
_D = 128
_B = 16384
_NC = 2
_NS = 16
_NW = _NC * _NS
_BPW = _B // _NW
_CH = 128
_NCHUNK = _BPW // _CH


def _sc_body(x_hbm, lab_hbm, cen_hbm, out_hbm, idx_v, c_v, x_v, dist_v, sem):
    wid = lax.axis_index("s") * _NC + lax.axis_index("c")
    base = wid * _BPW
    for j in range(_NCHUNK):
        pltpu.sync_copy(lab_hbm.at[pl.ds(base + j * _CH, _CH)], idx_v.at[j])
    for j in range(_NCHUNK):
        gather = pltpu.async_copy(cen_hbm.at[idx_v.at[j]], c_v, sem)
        pltpu.sync_copy(x_hbm.at[pl.ds(base + j * _CH, _CH)], x_v)
        gather.wait()
        for g in range(_CH // 16):
            rows = lax.iota(jnp.int32, 16) + (g * 16)

            def fbody(t, acc, rows=rows):
                for k in range(8):
                    fv = jnp.full((16,), t * 8 + k, jnp.int32)
                    xg = plsc.load_gather(x_v, [rows, fv])
                    cg = plsc.load_gather(c_v, [rows, fv])
                    d = xg - cg
                    acc = acc + d * d
                return acc

            acc = lax.fori_loop(0, _D // 8, fbody, jnp.zeros((16,), jnp.float32))
            dist_v[pl.ds(j * _CH + g * 16, 16)] = acc
    pltpu.sync_copy(dist_v, out_hbm.at[pl.ds(base, _BPW)])


_sc_dist = functools.partial(
    pl.kernel,
    out_type=jax.ShapeDtypeStruct((_B,), jnp.float32),
    mesh=plsc.VectorSubcoreMesh(core_axis_name="c", subcore_axis_name="s"),
    compiler_params=pltpu.CompilerParams(needs_layout_passes=False),
    scratch_types=[
        pltpu.VMEM((_NCHUNK, _CH), jnp.int32),
        pltpu.VMEM((_CH, _D), jnp.float32),
        pltpu.VMEM((_CH, _D), jnp.float32),
        pltpu.VMEM((_BPW,), jnp.float32),
        pltpu.SemaphoreType.DMA,
    ],
)(_sc_body)


def _tc_finish(d_ref, o_ref):
    d = jnp.clip(d_ref[...], 1e-12, 1e12)
    o_ref[...] = jnp.sum(d, axis=(0, 1), keepdims=True) * (1.0 / _B)


_finish = pl.pallas_call(
    _tc_finish,
    out_shape=jax.ShapeDtypeStruct((1, 1), jnp.float32),
)


def kernel(x, labels, centers):
    dists = _sc_dist(x, labels.astype(jnp.int32), centers)
    return _finish(dists.reshape(_B // _D, _D))[0, 0]


# speedup vs baseline: 2.6295x; 2.6295x over previous
import functools

import jax
import jax.numpy as jnp
from jax import lax
from jax.experimental import pallas as pl
from jax.experimental.pallas import tpu as pltpu
from jax.experimental.pallas import tpu_sc as plsc

_NUM_CLASSES = 100000
_D = 128
_B = 16384
_NC = 2
_NS = 16
_NW = _NC * _NS
_BPW = _B // _NW
_CH = 128
_NCHUNK = _BPW // _CH


def _sc_body(x_hbm, lab_hbm, cen_hbm, out_hbm, idx_v, c_v, x_v, dist_v,
             sem_i, sem_c, sem_x):
    wid = lax.axis_index("s") * _NC + lax.axis_index("c")
    base = wid * _BPW
    idx_cps = [
        pltpu.async_copy(lab_hbm.at[pl.ds(base + j * _CH, _CH)], idx_v.at[j],
                         sem_i)
        for j in range(_NCHUNK)
    ]
    idx_cps[0].wait()

    def start_chunk(j):
        p = j % 2
        c_cp = pltpu.async_copy(cen_hbm.at[idx_v.at[j]], c_v.at[p], sem_c)
        x_cp = pltpu.async_copy(x_hbm.at[pl.ds(base + j * _CH, _CH)],
                                x_v.at[p], sem_x)
        return c_cp, x_cp

    inflight = start_chunk(0)
    for j in range(_NCHUNK):
        if j + 1 < _NCHUNK:
            idx_cps[j + 1].wait()
            nxt = start_chunk(j + 1)
        c_cp, x_cp = inflight
        c_cp.wait()
        x_cp.wait()
        p = j % 2
        last_lane = lax.iota(jnp.int32, 16) == 15

        def row_body(r, carry, p=p, j=j, last_lane=last_lane):
            parts = []
            for k in range(8):
                xk = x_v[p, r, pl.ds(k * 16, 16)]
                ck = c_v[p, r, pl.ds(k * 16, 16)]
                d = xk - ck
                parts.append(d * d)
            s01 = parts[0] + parts[1]
            s23 = parts[2] + parts[3]
            s45 = parts[4] + parts[5]
            s67 = parts[6] + parts[7]
            s = (s01 + s23) + (s45 + s67)
            tot = plsc.cumsum(s)
            plsc.store_scatter(dist_v, [jnp.full((16,), j * _CH + r, jnp.int32)],
                               tot, mask=last_lane)
            return carry

        lax.fori_loop(0, _CH, row_body, 0, unroll=2)
        if j + 1 < _NCHUNK:
            inflight = nxt
    pltpu.sync_copy(dist_v, out_hbm.at[pl.ds(base, _BPW)])


_sc_dist = functools.partial(
    pl.kernel,
    out_type=jax.ShapeDtypeStruct((_B,), jnp.float32),
    mesh=plsc.VectorSubcoreMesh(core_axis_name="c", subcore_axis_name="s"),
    compiler_params=pltpu.CompilerParams(needs_layout_passes=False),
    scratch_types=[
        pltpu.VMEM((_NCHUNK, _CH), jnp.int32),
        pltpu.VMEM((2, _CH, _D), jnp.float32),
        pltpu.VMEM((2, _CH, _D), jnp.float32),
        pltpu.VMEM((_BPW,), jnp.float32),
        pltpu.SemaphoreType.DMA,
        pltpu.SemaphoreType.DMA,
        pltpu.SemaphoreType.DMA,
    ],
)(_sc_body)


def _tc_finish(d_ref, o_ref):
    d = jnp.clip(d_ref[...], 1e-12, 1e12)
    o_ref[...] = jnp.sum(d, axis=(0, 1), keepdims=True) * (1.0 / _B)


_finish = pl.pallas_call(
    _tc_finish,
    out_shape=jax.ShapeDtypeStruct((1, 1), jnp.float32),
)


def kernel(x, labels, centers):
    dists = _sc_dist(x, labels.astype(jnp.int32), centers)
    return _finish(dists.reshape(_B // _D, _D))[0, 0]


# speedup vs baseline: 3.1304x; 1.1905x over previous
import functools

import jax
import jax.numpy as jnp
from jax import lax
from jax.experimental import pallas as pl
from jax.experimental.pallas import tpu as pltpu
from jax.experimental.pallas import tpu_sc as plsc

_NUM_CLASSES = 100000
_D = 128
_B = 16384
_NC = 2
_NS = 16
_NW = _NC * _NS
_BPW = _B // _NW
_CH = 128
_NCHUNK = _BPW // _CH


def _sc_body(x_hbm, lab_hbm, cen_hbm, out_hbm, idx_v, c_v, x_v, acc_v,
             sem_i, sem_c, sem_x):
    wid = lax.axis_index("s") * _NC + lax.axis_index("c")
    base = wid * _BPW
    idx_cps = [
        pltpu.async_copy(lab_hbm.at[pl.ds(base + j * _CH, _CH)], idx_v.at[j],
                         sem_i)
        for j in range(_NCHUNK)
    ]
    idx_cps[0].wait()

    def start_chunk(j):
        p = j % 2
        c_cp = pltpu.async_copy(cen_hbm.at[idx_v.at[j]], c_v.at[p], sem_c)
        x_cp = pltpu.async_copy(x_hbm.at[pl.ds(base + j * _CH, _CH)],
                                x_v.at[p], sem_x)
        return c_cp, x_cp

    inflight = start_chunk(0)
    last_lane = lax.iota(jnp.int32, 16) == 15
    zeros = jnp.zeros((16,), jnp.float32)
    acc = zeros
    for j in range(_NCHUNK):
        if j + 1 < _NCHUNK:
            idx_cps[j + 1].wait()
            nxt = start_chunk(j + 1)
        c_cp, x_cp = inflight
        c_cp.wait()
        x_cp.wait()
        p = j % 2

        @plsc.parallel_loop(0, _CH, unroll=4, carry=acc)
        def row_body(r, acc, p=p, last_lane=last_lane, zeros=zeros):
            parts = []
            for k in range(8):
                xk = x_v[p, r, pl.ds(k * 16, 16)]
                ck = c_v[p, r, pl.ds(k * 16, 16)]
                d = xk - ck
                parts.append(d * d)
            s01 = parts[0] + parts[1]
            s23 = parts[2] + parts[3]
            s45 = parts[4] + parts[5]
            s67 = parts[6] + parts[7]
            s = (s01 + s23) + (s45 + s67)
            tot = plsc.cumsum(s)
            tot = jnp.minimum(jnp.maximum(tot, 1e-12), 1e12)
            return acc + jnp.where(last_lane, tot, zeros)

        acc = row_body
        if j + 1 < _NCHUNK:
            inflight = nxt
    acc_v[...] = acc
    pltpu.sync_copy(acc_v, out_hbm.at[wid])


_sc_dist = functools.partial(
    pl.kernel,
    out_type=jax.ShapeDtypeStruct((_NW, 16), jnp.float32),
    mesh=plsc.VectorSubcoreMesh(core_axis_name="c", subcore_axis_name="s"),
    compiler_params=pltpu.CompilerParams(needs_layout_passes=False),
    scratch_types=[
        pltpu.VMEM((_NCHUNK, _CH), jnp.int32),
        pltpu.VMEM((2, _CH, _D), jnp.float32),
        pltpu.VMEM((2, _CH, _D), jnp.float32),
        pltpu.VMEM((16,), jnp.float32),
        pltpu.SemaphoreType.DMA,
        pltpu.SemaphoreType.DMA,
        pltpu.SemaphoreType.DMA,
    ],
)(_sc_body)


def _tc_finish(d_ref, o_ref):
    o_ref[...] = jnp.sum(d_ref[...], axis=(0, 1), keepdims=True) * (1.0 / _B)


_finish = pl.pallas_call(
    _tc_finish,
    out_shape=jax.ShapeDtypeStruct((1, 1), jnp.float32),
)


def kernel(x, labels, centers):
    partials = _sc_dist(x, labels.astype(jnp.int32), centers)
    return _finish(partials)[0, 0]
